# baseline (speedup 1.0000x reference)
import jax
import jax.numpy as jnp
from jax import lax
from jax.experimental import pallas as pl
from jax.experimental.pallas import tpu as pltpu
from jax.experimental.pallas import tpu_sc as plsc

B, H, S, D = 8, 16, 4096, 128
Q = 16
BH = B * H

NC, NS = 2, 16
NW = NC * NS
BH_PER_W = BH // NW


def _tc_copy_body(kc_ref, vc_ref, ko_ref, vo_ref):
    ko_ref[...] = kc_ref[...]
    vo_ref[...] = vc_ref[...]


def _tc_copy(kc, vc):
    cache_spec = pl.BlockSpec((1, S, D), lambda i: (i, 0, 0))
    return pl.pallas_call(
        _tc_copy_body,
        grid=(BH,),
        in_specs=[cache_spec, cache_spec],
        out_specs=[cache_spec, cache_spec],
        out_shape=[
            jax.ShapeDtypeStruct((BH, S, D), jnp.float32),
            jax.ShapeDtypeStruct((BH, S, D), jnp.float32),
        ],
        compiler_params=pltpu.CompilerParams(
            dimension_semantics=("arbitrary",),
        ),
    )(kc, vc)


def _sc_scatter_body(pos_hbm, kval_hbm, vval_hbm, ko_ref, vo_ref,
                     pos_v, src_v, dst_v, krows, vrows, sem):
    wid = lax.axis_index("s") * NC + lax.axis_index("c")

    pltpu.sync_copy(pos_hbm, pos_v)
    pos = pos_v[...]
    iota = lax.iota(jnp.int32, 16)
    m = iota
    for r in range(1, Q):
        pos_r = jnp.take_along_axis(pos, jnp.full((Q,), r, jnp.int32), axis=0)
        m = jnp.where(pos == pos_r, r, m)

    for j in range(BH_PER_W):
        bh = wid * BH_PER_W + j
        src_v[pl.ds(j * Q, Q)] = bh * Q + m
        dst_v[pl.ds(j * Q, Q)] = bh * S + pos

    pltpu.async_copy(kval_hbm.at[src_v], krows, sem).wait()
    pltpu.async_copy(vval_hbm.at[src_v], vrows, sem).wait()
    pltpu.async_copy(krows, ko_ref.at[dst_v], sem).wait()
    pltpu.async_copy(vrows, vo_ref.at[dst_v], sem).wait()


_sc_scatter = pl.kernel(
    _sc_scatter_body,
    out_type=(),
    mesh=plsc.VectorSubcoreMesh(core_axis_name="c", subcore_axis_name="s"),
    scratch_types=[
        pltpu.VMEM((Q,), jnp.int32),
        pltpu.VMEM((BH_PER_W * Q,), jnp.int32),
        pltpu.VMEM((BH_PER_W * Q,), jnp.int32),
        pltpu.VMEM((BH_PER_W * Q, D), jnp.float32),
        pltpu.VMEM((BH_PER_W * Q, D), jnp.float32),
        pltpu.SemaphoreType.DMA,
    ],
)


def kernel(input_pos, k_val, v_val, k_cache, v_cache):
    kc = k_cache.reshape(BH, S, D)
    vc = v_cache.reshape(BH, S, D)
    ko, vo = _tc_copy(kc, vc)
    ko_ref = jax.new_ref(ko.reshape(BH * S, D))
    vo_ref = jax.new_ref(vo.reshape(BH * S, D))
    _sc_scatter(input_pos, k_val.reshape(BH * Q, D), v_val.reshape(BH * Q, D),
                ko_ref, vo_ref)
    return ko_ref[...].reshape(B, H, S, D), vo_ref[...].reshape(B, H, S, D)
